# Initial kernel scaffold; baseline (speedup 1.0000x reference)
import jax
import jax.numpy as jnp
from jax.experimental import pallas as pl


def kernel(x, Wg, W1, b1, W2, b2):
    raise NotImplementedError("write your pallas kernel here")


# speedup vs baseline: 2.4428x; 2.4428x over previous
import functools
import math

import jax
import jax.numpy as jnp
from jax.experimental import pallas as pl
from jax.experimental.pallas import tpu as pltpu

_K = 2


def _routing_kernel(x_ref, wg_ref, ti_ref, tw_ref, cnt_ref, lb_ref, ovf_ref,
                    *, n_exp, cap):
    x = x_ref[...]
    wg = wg_ref[...]
    logits = jax.lax.dot_general(
        x, wg, (((1,), (1,)), ((), ())),
        preferred_element_type=jnp.float32)
    m = jnp.max(logits, axis=1, keepdims=True)
    ex = jnp.exp(logits - m)
    probs = ex / jnp.sum(ex, axis=1, keepdims=True)

    lane = jax.lax.broadcasted_iota(jnp.int32, probs.shape, 1)
    big = jnp.int32(10**6)
    m1 = jnp.max(probs, axis=1, keepdims=True)
    i1 = jnp.min(jnp.where(probs == m1, lane, big), axis=1, keepdims=True)
    p2 = jnp.where(lane == i1, -1.0, probs)
    m2 = jnp.max(p2, axis=1, keepdims=True)
    i2 = jnp.min(jnp.where(p2 == m2, lane, big), axis=1, keepdims=True)

    s = jnp.maximum(m1 + m2, 1e-9)
    w1 = m1 / s
    w2 = m2 / s

    ti_ref[...] = jnp.concatenate([i1, i2], axis=1)
    tw_ref[...] = jnp.concatenate([w1, w2], axis=1)

    erow = jax.lax.broadcasted_iota(jnp.int32, (1, n_exp), 1)
    cnts = (jnp.sum((i1 == erow).astype(jnp.float32), axis=0, keepdims=True)
            + jnp.sum((i2 == erow).astype(jnp.float32), axis=0, keepdims=True))
    cnt_ref[...] = cnts

    bt = x.shape[0]
    expected = jnp.float32(bt * _K) / n_exp
    lb_ref[...] = jnp.mean((cnts - expected) ** 2).reshape(1, 1) / (expected ** 2)

    kept_total = jnp.sum(jnp.minimum(cnts, jnp.float32(cap)))
    ovf_ref[...] = ((jnp.float32(bt * _K) - kept_total)
                    / jnp.float32(bt * _K)).reshape(1, 1)


def _expert_kernel(tok_ref, x_ref, w1_ref, b1_ref, w2_ref, b2_ref, gw_ref,
                   y_ref, xg_ref, yg_ref, *, cap, unroll):
    e = pl.program_id(0)
    h = pl.program_id(1)
    nh = pl.num_programs(1)
    base_off = e * cap

    @pl.when((e == 0) & (h == 0))
    def _():
        y_ref[...] = jnp.zeros_like(y_ref)

    @pl.when(h == 0)
    def _():
        def gather_body(j, carry):
            b = j * unroll
            for u in range(unroll):
                t = tok_ref[base_off + b + u]
                xg_ref[pl.ds(b + u, 1), :] = x_ref[pl.ds(t, 1), :]
            return carry

        jax.lax.fori_loop(0, cap // unroll, gather_body, 0)

    hid = jnp.dot(xg_ref[...], w1_ref[0], preferred_element_type=jnp.float32)
    hid = jnp.maximum(hid + b1_ref[0], 0.0)
    part = jnp.dot(hid, w2_ref[0], preferred_element_type=jnp.float32)

    @pl.when(h == 0)
    def _():
        yg_ref[...] = part

    @pl.when(h != 0)
    def _():
        yg_ref[...] = yg_ref[...] + part

    @pl.when(h == nh - 1)
    def _():
        yg_ref[...] = (yg_ref[...] + b2_ref[0]) * gw_ref[0]

        def scatter_body(j, carry):
            b = j * unroll
            for u in range(unroll):
                t = tok_ref[base_off + b + u]
                y_ref[pl.ds(t, 1), :] = (y_ref[pl.ds(t, 1), :]
                                         + yg_ref[pl.ds(b + u, 1), :])
            return carry

        jax.lax.fori_loop(0, cap // unroll, scatter_body, 0)


def kernel(x, Wg, W1, b1, W2, b2):
    B, T, D = x.shape
    N, _, DH = W1.shape
    BT = B * T
    C = math.ceil(1.25 * (BT * _K) / N)

    x_flat = x.reshape(BT, D)

    ti, tw, cnts, lb, ovf = pl.pallas_call(
        functools.partial(_routing_kernel, n_exp=N, cap=C),
        out_shape=[
            jax.ShapeDtypeStruct((BT, _K), jnp.int32),
            jax.ShapeDtypeStruct((BT, _K), jnp.float32),
            jax.ShapeDtypeStruct((1, N), jnp.float32),
            jax.ShapeDtypeStruct((1, 1), jnp.float32),
            jax.ShapeDtypeStruct((1, 1), jnp.float32),
        ],
    )(x_flat, Wg)

    e_flat = ti.reshape(-1)
    w_flat = tw.reshape(-1)
    o1 = jnp.argsort(-w_flat, stable=True)
    o2 = jnp.argsort(e_flat[o1], stable=True)
    order = o1[o2]
    e_s = e_flat[order]
    cnt_i = cnts.reshape(-1).astype(jnp.int32)
    starts = jnp.cumsum(cnt_i) - cnt_i
    rank_s = jnp.arange(BT * _K, dtype=jnp.int32) - starts[e_s]
    kept_s = rank_s < C
    slot = jnp.where(kept_s, e_s * C + rank_s, N * C)
    tok_s = (order // _K).astype(jnp.int32)
    w_s = jnp.where(kept_s, w_flat[order], 0.0)
    tok_list = jnp.zeros((N * C + 1,), jnp.int32).at[slot].set(tok_s)[:N * C]
    w_list = jnp.zeros((N * C + 1,), jnp.float32).at[slot].set(w_s)[:N * C]

    rank_flat = jnp.zeros((BT * _K,), jnp.int32).at[order].set(rank_s)
    contrib = (w_flat * (rank_flat < C)).reshape(BT, _K).sum(axis=1)
    pmask = (contrib <= 1e-12).astype(jnp.float32)[:, None]

    gw = w_list.reshape(N, C, 1)

    unroll = 8
    nh = 4
    dh = DH // nh
    y_flat = pl.pallas_call(
        functools.partial(_expert_kernel, cap=C, unroll=unroll),
        grid_spec=pltpu.PrefetchScalarGridSpec(
            num_scalar_prefetch=1,
            grid=(N, nh),
            in_specs=[
                pl.BlockSpec((BT, D), lambda e, h, tok: (0, 0)),
                pl.BlockSpec((1, D, dh), lambda e, h, tok: (e, 0, h)),
                pl.BlockSpec((1, 1, dh), lambda e, h, tok: (e, 0, h)),
                pl.BlockSpec((1, dh, D), lambda e, h, tok: (e, h, 0)),
                pl.BlockSpec((1, 1, D), lambda e, h, tok: (e, 0, 0)),
                pl.BlockSpec((1, C, 1), lambda e, h, tok: (e, 0, 0)),
            ],
            out_specs=pl.BlockSpec((BT, D), lambda e, h, tok: (0, 0)),
            scratch_shapes=[
                pltpu.VMEM((C, D), jnp.float32),
                pltpu.VMEM((C, D), jnp.float32),
            ],
        ),
        out_shape=jax.ShapeDtypeStruct((BT, D), jnp.float32),
    )(tok_list, x_flat, W1, b1.reshape(N, 1, DH), W2, b2.reshape(N, 1, D), gw)

    y_flat = jnp.where(pmask > 0, x_flat, y_flat)
    return (y_flat.reshape(B, T, D), lb.reshape(()), ovf.reshape(()))
